# Initial kernel scaffold; baseline (speedup 1.0000x reference)
import jax
import jax.numpy as jnp
from jax.experimental import pallas as pl


def kernel(input, weight):
    raise NotImplementedError("write your pallas kernel here")


# speedup vs baseline: 1.1129x; 1.1129x over previous
import functools

import jax
import jax.numpy as jnp
from jax import lax
from jax.experimental import pallas as pl
from jax.experimental.pallas import tpu as pltpu
from jax.experimental.pallas import tpu_sc as plsc

_info = plsc.get_sparse_core_info()
_NC, _NS = _info.num_cores, _info.num_subcores
_NW = _NC * _NS

_CHUNK = 128
_NSLOT = 8
_K = 4


@functools.partial(jax.jit, static_argnames=("n_chunks", "dim"))
def _sc_gather(idx, weight, *, n_chunks, dim):
    n = _NW * n_chunks * _CHUNK
    per_w = n_chunks * _CHUNK
    assert n_chunks % _NSLOT == 0
    n_groups = n_chunks // _NSLOT
    mesh = plsc.VectorSubcoreMesh(core_axis_name="c", subcore_axis_name="s")

    @functools.partial(
        pl.kernel,
        mesh=mesh,
        out_type=jax.ShapeDtypeStruct((n, dim), jnp.float32),
        scratch_types=[
            pltpu.VMEM((n_chunks, _CHUNK), jnp.int32),
            pltpu.VMEM((_NSLOT, _CHUNK, dim), jnp.float32),
            pltpu.SemaphoreType.DMA((_NSLOT,)),
            pltpu.SemaphoreType.DMA((_NSLOT,)),
        ],
        compiler_params=pltpu.CompilerParams(use_tc_tiling_on_sc=False),
    )
    def k(idx_hbm, tab_hbm, out_hbm, idx_v, rows_v, gsem, wsem):
        wid = lax.axis_index("s") * _NC + lax.axis_index("c")
        base = wid * per_w
        pltpu.sync_copy(idx_hbm.at[wid], idx_v)

        def fire_gather(c, slot):
            pltpu.async_copy(tab_hbm.at[idx_v.at[c]], rows_v.at[slot],
                             gsem.at[slot])

        def wait_gather(c, slot):
            pltpu.make_async_copy(tab_hbm.at[idx_v.at[c]], rows_v.at[slot],
                                  gsem.at[slot]).wait()

        def fire_write(c, slot):
            pltpu.async_copy(rows_v.at[slot],
                             out_hbm.at[pl.ds(base + c * _CHUNK, _CHUNK)],
                             wsem.at[slot])

        def wait_write(c, slot):
            pltpu.make_async_copy(rows_v.at[slot],
                                  out_hbm.at[pl.ds(base + c * _CHUNK, _CHUNK)],
                                  wsem.at[slot]).wait()

        for b in range(_K):
            fire_gather(b, b)

        def group(g, _):
            for b in range(_NSLOT):
                c = g * _NSLOT + b
                nxt = c + _K
                ns = (b + _K) % _NSLOT

                @pl.when(jnp.logical_and(nxt < n_chunks, nxt - _NSLOT >= 0))
                def _():
                    wait_write(nxt - _NSLOT, ns)

                @pl.when(nxt < n_chunks)
                def _():
                    fire_gather(nxt, ns)

                wait_gather(c, b)
                fire_write(c, b)
            return 0

        lax.fori_loop(0, n_groups, group, 0)

        for b in range(_NSLOT):
            c = n_chunks - _NSLOT + b
            wait_write(c, c % _NSLOT)

    return k(idx, weight)


def kernel(input, weight):
    B, L = input.shape
    V, D = weight.shape
    n = B * L
    assert n % (_NW * _CHUNK) == 0
    n_chunks = n // (_NW * _CHUNK)
    idx = input.reshape(_NW, n_chunks, _CHUNK)
    out = _sc_gather(idx, weight, n_chunks=n_chunks, dim=D)
    return out.reshape(B, L, D)


# speedup vs baseline: 1.7660x; 1.5869x over previous
import functools

import jax
import jax.numpy as jnp
from jax import lax
from jax.experimental import pallas as pl
from jax.experimental.pallas import tpu as pltpu
from jax.experimental.pallas import tpu_sc as plsc

_info = plsc.get_sparse_core_info()
_NC, _NS = _info.num_cores, _info.num_subcores
_NW = _NC * _NS

_GB = 8
_KG = 4


@functools.partial(jax.jit, static_argnames=("batch", "hist", "dim"))
def _sc_embed(idx, weight, *, batch, hist, dim):
    rows_per_w = batch // _NW
    n_groups = rows_per_w // _GB
    assert n_groups % 2 == 0 and n_groups >= 6
    mesh = plsc.VectorSubcoreMesh(core_axis_name="c", subcore_axis_name="s")

    @functools.partial(
        pl.kernel,
        mesh=mesh,
        out_type=jax.ShapeDtypeStruct((batch, hist, dim), jnp.float32),
        scratch_types=[
            pltpu.VMEM((2, _GB, hist), jnp.int32),
            pltpu.VMEM((_GB, hist, dim), jnp.float32),
            pltpu.SemaphoreType.DMA((2,)),
            pltpu.SemaphoreType.DMA((_GB,)),
            pltpu.SemaphoreType.DMA((_GB,)),
        ],
        compiler_params=pltpu.CompilerParams(use_tc_tiling_on_sc=False),
    )
    def k(idx_hbm, tab_hbm, out_hbm, idxg, rows_v, isem, gsem, wsem):
        wid = lax.axis_index("s") * _NC + lax.axis_index("c")
        r0 = wid * rows_per_w

        def stage(g, par):
            pltpu.async_copy(idx_hbm.at[pl.ds(r0 + g * _GB, _GB)],
                             idxg.at[par], isem.at[par])

        def wait_stage(g, par):
            pltpu.make_async_copy(idx_hbm.at[pl.ds(r0 + g * _GB, _GB)],
                                  idxg.at[par], isem.at[par]).wait()

        def fire_gather(par, j, s):
            pltpu.async_copy(tab_hbm.at[idxg.at[par].at[j]], rows_v.at[s],
                             gsem.at[s])

        def wait_gather(par, j, s):
            pltpu.make_async_copy(tab_hbm.at[idxg.at[par].at[j]],
                                  rows_v.at[s], gsem.at[s]).wait()

        def fire_write(c, s):
            pltpu.async_copy(rows_v.at[s], out_hbm.at[r0 + c], wsem.at[s])

        def wait_write(c, s):
            pltpu.make_async_copy(rows_v.at[s], out_hbm.at[r0 + c],
                                  wsem.at[s]).wait()

        def step(c, b, g, par, *, do_wait_write=True, do_stage=True,
                 do_gather=True):
            if b == 0 and do_stage:
                stage(g + 1, 1 - par)
            if do_gather:
                bx = (b + _KG) % _GB
                parx = (par + (b + _KG) // _GB) % 2
                if do_wait_write:
                    wait_write(c + _KG - _GB, bx)
                if bx == 0:
                    wait_stage(g + 1, parx)
                fire_gather(parx, bx, bx)
            wait_gather(par, b, b)
            fire_write(c, b)

        stage(0, 0)
        wait_stage(0, 0)
        for b in range(_KG):
            fire_gather(0, b, b)
        for cc in range(2 * _GB):
            step(cc, cc % _GB, cc // _GB, (cc // _GB) % 2,
                 do_wait_write=(cc >= _KG))

        def pair(gg, _):
            cbase = gg * (2 * _GB)
            for off in range(2 * _GB):
                step(cbase + off, off % _GB, 2 * gg + off // _GB,
                     off // _GB)
            return 0

        lax.fori_loop(1, n_groups // 2 - 1, pair, 0)

        for off in range(2 * _GB):
            cc = rows_per_w - 2 * _GB + off
            g = n_groups - 2 + off // _GB
            step(cc, off % _GB, g, g % 2,
                 do_stage=(g + 1 < n_groups),
                 do_gather=(cc + _KG < rows_per_w))

        for b in range(_GB):
            c = rows_per_w - _GB + b
            wait_write(c, b)

    return k(idx, weight)


def kernel(input, weight):
    B, L = input.shape
    V, D = weight.shape
    assert B % (_NW * 2 * _GB) == 0
    return _sc_embed(input, weight, batch=B, hist=L, dim=D)
